# Initial kernel scaffold; baseline (speedup 1.0000x reference)
import jax
import jax.numpy as jnp
from jax.experimental import pallas as pl


def kernel(x, W1, W2, W3, W4, W5):
    raise NotImplementedError("write your pallas kernel here")


# speedup vs baseline: 2.5628x; 2.5628x over previous
import functools

import jax
import jax.numpy as jnp
from jax import lax
from jax.experimental import pallas as pl

EPS = 1e-5
K = 20
B = 8
N = 1024
CNT_EDGE = float(B * N * K)
CNT_PTS = float(B * N)
NEG_INF = float("-inf")


def _lrelu(x):
    return jnp.where(x >= 0, x, 0.2 * x)


def _norm_act(h, psum, pqsum, cnt):
    m = jnp.sum(psum[:, 0, :], axis=0) / cnt
    v = jnp.sum(pqsum[:, 0, :], axis=0) / cnt - m * m
    sd = jnp.sqrt(v + EPS)
    return _lrelu((h - m) / sd)


def _l1_body(x_ref, w_ref, hmax_ref, ps_ref, pq_ref):
    b = pl.program_id(0)
    k = pl.program_id(1)
    xk = x_ref[0, 0]
    w = w_ref[...]
    h = jnp.dot(xk, w, preferred_element_type=jnp.float32)

    @pl.when(k == 0)
    def _():
        ps_ref[...] = jnp.zeros_like(ps_ref)
        pq_ref[...] = jnp.zeros_like(pq_ref)
        hmax_ref[0] = jnp.full_like(hmax_ref[0], NEG_INF)

    ps_ref[...] += jnp.sum(h, axis=0, keepdims=True)
    pq_ref[...] += jnp.sum(h * h, axis=0, keepdims=True)
    hmax_ref[0] = jnp.maximum(hmax_ref[0], h)


def _layer1(x, w1t):
    return pl.pallas_call(
        _l1_body,
        grid=(B, K),
        in_specs=[
            pl.BlockSpec((1, 1, N, 6), lambda b, k: (b, k, 0, 0)),
            pl.BlockSpec((6, 64), lambda b, k: (0, 0)),
        ],
        out_specs=[
            pl.BlockSpec((1, N, 64), lambda b, k: (b, 0, 0)),
            pl.BlockSpec((1, 1, 64), lambda b, k: (b, 0, 0)),
            pl.BlockSpec((1, 1, 64), lambda b, k: (b, 0, 0)),
        ],
        out_shape=[
            jax.ShapeDtypeStruct((B, N, 64), jnp.float32),
            jax.ShapeDtypeStruct((B, 1, 64), jnp.float32),
            jax.ShapeDtypeStruct((B, 1, 64), jnp.float32),
        ],
    )(x, w1t)


RB = 128
NRB = N // RB


def _graph_body(c_in, c_out, hp_ref, ps_ref, pq_ref, wt_ref,
                hmax_ref, pso_ref, pqo_ref):
    b = pl.program_id(0)
    xin = _norm_act(hp_ref[0], ps_ref[...], pq_ref[...], CNT_EDGE)
    wt = wt_ref[...]
    xx = jnp.sum(xin * xin, axis=1)
    xrow = xx[None, :]
    iota = lax.broadcasted_iota(jnp.int32, (RB, N), 1)

    mx_parts = []
    p_parts = []
    q_parts = []
    for rb in range(NRB):
        sl = slice(rb * RB, (rb + 1) * RB)
        xb = xin[sl]
        gb = lax.dot_general(xb, xin, (((1,), (1,)), ((), ())),
                             preferred_element_type=jnp.float32)
        inner = -2.0 * gb
        db = (-xx[sl][:, None] - inner) - xrow

        def step(_, carry):
            d, mxb, sb, qb = carry
            v = jnp.max(d, axis=1, keepdims=True)
            eq = d == v
            cand = jnp.min(jnp.where(eq, iota, N + 1), axis=1, keepdims=True)
            sel = iota == cand
            gx = jnp.dot(sel.astype(jnp.float32), xin,
                         preferred_element_type=jnp.float32,
                         precision=lax.Precision.HIGHEST)
            feat = jnp.concatenate([gx - xb, xb], axis=1)
            h = jnp.dot(feat, wt, preferred_element_type=jnp.float32)
            return (jnp.where(sel, NEG_INF, d), jnp.maximum(mxb, h),
                    sb + h, qb + h * h)

        init = (db,
                jnp.full((RB, c_out), NEG_INF, jnp.float32),
                jnp.zeros((RB, c_out), jnp.float32),
                jnp.zeros((RB, c_out), jnp.float32))
        _, mxb, sb, qb = lax.fori_loop(0, K, step, init)
        mx_parts.append(mxb)
        p_parts.append(jnp.sum(sb, axis=0))
        q_parts.append(jnp.sum(qb, axis=0))

    hmax_ref[0] = jnp.concatenate(mx_parts, axis=0)
    pso_ref[0] = jnp.sum(jnp.stack(p_parts, 0), axis=0, keepdims=True)
    pqo_ref[0] = jnp.sum(jnp.stack(q_parts, 0), axis=0, keepdims=True)


def _graph_layer(hmax, ps, pq, w, c_in, c_out):
    wt = jnp.transpose(w)
    return pl.pallas_call(
        functools.partial(_graph_body, c_in, c_out),
        grid=(B,),
        in_specs=[
            pl.BlockSpec((1, N, c_in), lambda b: (b, 0, 0)),
            pl.BlockSpec((B, 1, c_in), lambda b: (0, 0, 0)),
            pl.BlockSpec((B, 1, c_in), lambda b: (0, 0, 0)),
            pl.BlockSpec((2 * c_in, c_out), lambda b: (0, 0)),
        ],
        out_specs=[
            pl.BlockSpec((1, N, c_out), lambda b: (b, 0, 0)),
            pl.BlockSpec((1, 1, c_out), lambda b: (b, 0, 0)),
            pl.BlockSpec((1, 1, c_out), lambda b: (b, 0, 0)),
        ],
        out_shape=[
            jax.ShapeDtypeStruct((B, N, c_out), jnp.float32),
            jax.ShapeDtypeStruct((B, 1, c_out), jnp.float32),
            jax.ShapeDtypeStruct((B, 1, c_out), jnp.float32),
        ],
    )(hmax, ps, pq, wt)


def _l5_body(h1_ref, p1_ref, q1_ref, h2_ref, p2_ref, q2_ref,
             h3_ref, p3_ref, q3_ref, h4_ref, p4_ref, q4_ref,
             w_ref, hmax_ref, ps_ref, pq_ref):
    b = pl.program_id(0)
    x1 = _norm_act(h1_ref[0], p1_ref[...], q1_ref[...], CNT_EDGE)
    x2 = _norm_act(h2_ref[0], p2_ref[...], q2_ref[...], CNT_EDGE)
    x3 = _norm_act(h3_ref[0], p3_ref[...], q3_ref[...], CNT_EDGE)
    x4 = _norm_act(h4_ref[0], p4_ref[...], q4_ref[...], CNT_EDGE)
    xc = jnp.concatenate([x1, x2, x3, x4], axis=1)
    h = jnp.dot(xc, w_ref[...], preferred_element_type=jnp.float32)
    hmax_ref[0] = jnp.max(h, axis=0, keepdims=True)
    ps_ref[0] = jnp.sum(h, axis=0, keepdims=True)
    pq_ref[0] = jnp.sum(h * h, axis=0, keepdims=True)


def _layer5(parts, w5):
    w5t = jnp.transpose(w5)
    args = []
    for hm, ps, pq in parts:
        args += [hm, ps, pq]
    args.append(w5t)
    in_specs = []
    for _, ps, _ in parts:
        c = ps.shape[2]
        in_specs += [
            pl.BlockSpec((1, N, c), lambda b: (b, 0, 0)),
            pl.BlockSpec((B, 1, c), lambda b, c=c: (0, 0, 0)),
            pl.BlockSpec((B, 1, c), lambda b, c=c: (0, 0, 0)),
        ]
    in_specs.append(pl.BlockSpec((512, 1024), lambda b: (0, 0)))
    return pl.pallas_call(
        _l5_body,
        grid=(B,),
        in_specs=in_specs,
        out_specs=[
            pl.BlockSpec((1, 1, 1024), lambda b: (b, 0, 0)),
            pl.BlockSpec((1, 1, 1024), lambda b: (b, 0, 0)),
            pl.BlockSpec((1, 1, 1024), lambda b: (b, 0, 0)),
        ],
        out_shape=[
            jax.ShapeDtypeStruct((B, 1, 1024), jnp.float32),
            jax.ShapeDtypeStruct((B, 1, 1024), jnp.float32),
            jax.ShapeDtypeStruct((B, 1, 1024), jnp.float32),
        ],
    )(*args)


def _final_body(hm_ref, ps_ref, pq_ref, out_ref):
    out_ref[...] = _norm_act(hm_ref[:, 0, :], ps_ref[...], pq_ref[...],
                             CNT_PTS)


def _final(hmax5, ps5, pq5):
    return pl.pallas_call(
        _final_body,
        out_shape=jax.ShapeDtypeStruct((B, 1024), jnp.float32),
    )(hmax5, ps5, pq5)


def kernel(x, W1, W2, W3, W4, W5):
    hm1, ps1, pq1 = _layer1(jnp.transpose(x, (0, 3, 2, 1)), jnp.transpose(W1))
    hm2, ps2, pq2 = _graph_layer(hm1, ps1, pq1, W2, 64, 64)
    hm3, ps3, pq3 = _graph_layer(hm2, ps2, pq2, W3, 64, 128)
    hm4, ps4, pq4 = _graph_layer(hm3, ps3, pq3, W4, 128, 256)
    hm5, ps5, pq5 = _layer5(
        [(hm1, ps1, pq1), (hm2, ps2, pq2), (hm3, ps3, pq3), (hm4, ps4, pq4)],
        W5)
    out = _final(hm5, ps5, pq5)
    return out.reshape(B, 1024, 1)


# speedup vs baseline: 3.1633x; 1.2343x over previous
import functools

import jax
import jax.numpy as jnp
from jax import lax
from jax.experimental import pallas as pl

EPS = 1e-5
K = 20
B = 8
N = 1024
CNT_EDGE = float(B * N * K)
CNT_PTS = float(B * N)
NEG_INF = float("-inf")


def _lrelu(x):
    return jnp.where(x >= 0, x, 0.2 * x)


def _norm_act(h, psum, pqsum, cnt):
    m = jnp.sum(psum[:, 0, :], axis=0) / cnt
    v = jnp.sum(pqsum[:, 0, :], axis=0) / cnt - m * m
    sd = jnp.sqrt(v + EPS)
    return _lrelu((h - m) / sd)


def _l1_body(x_ref, w_ref, hmax_ref, ps_ref, pq_ref):
    b = pl.program_id(0)
    k = pl.program_id(1)
    xk = x_ref[0, 0]
    w = w_ref[...]
    h = jnp.dot(xk, w, preferred_element_type=jnp.float32)

    @pl.when(k == 0)
    def _():
        ps_ref[...] = jnp.zeros_like(ps_ref)
        pq_ref[...] = jnp.zeros_like(pq_ref)
        hmax_ref[0] = jnp.full_like(hmax_ref[0], NEG_INF)

    ps_ref[...] += jnp.sum(h, axis=0, keepdims=True)
    pq_ref[...] += jnp.sum(h * h, axis=0, keepdims=True)
    hmax_ref[0] = jnp.maximum(hmax_ref[0], h)


def _layer1(x, w1t):
    return pl.pallas_call(
        _l1_body,
        grid=(B, K),
        in_specs=[
            pl.BlockSpec((1, 1, N, 6), lambda b, k: (b, k, 0, 0)),
            pl.BlockSpec((6, 64), lambda b, k: (0, 0)),
        ],
        out_specs=[
            pl.BlockSpec((1, N, 64), lambda b, k: (b, 0, 0)),
            pl.BlockSpec((1, 1, 64), lambda b, k: (b, 0, 0)),
            pl.BlockSpec((1, 1, 64), lambda b, k: (b, 0, 0)),
        ],
        out_shape=[
            jax.ShapeDtypeStruct((B, N, 64), jnp.float32),
            jax.ShapeDtypeStruct((B, 1, 64), jnp.float32),
            jax.ShapeDtypeStruct((B, 1, 64), jnp.float32),
        ],
    )(x, w1t)


RB = 128
NRB = N // RB


def _graph_body(c_in, c_out, hp_ref, ps_ref, pq_ref, wt_ref,
                hmax_ref, pso_ref, pqo_ref):
    b = pl.program_id(0)
    xin = _norm_act(hp_ref[0], ps_ref[...], pq_ref[...], CNT_EDGE)
    wt = wt_ref[...]
    xh = xin.astype(jnp.bfloat16)
    r1 = xin - xh.astype(jnp.float32)
    xl = r1.astype(jnp.bfloat16)
    xl2 = (r1 - xl.astype(jnp.float32)).astype(jnp.bfloat16)
    xx = jnp.sum(xin * xin, axis=1)
    xrow = xx[None, :]
    iota = lax.broadcasted_iota(jnp.int32, (RB, N), 1)

    mx_parts = []
    p_parts = []
    q_parts = []
    for rb in range(NRB):
        sl = slice(rb * RB, (rb + 1) * RB)
        xb = xin[sl]
        gb = lax.dot_general(xb, xin, (((1,), (1,)), ((), ())),
                             preferred_element_type=jnp.float32)
        inner = -2.0 * gb
        db = (-xx[sl][:, None] - inner) - xrow

        def step(_, carry):
            d, mxb, sb, qb = carry
            v = jnp.max(d, axis=1, keepdims=True)
            eq = d == v
            cand = jnp.min(jnp.where(eq, iota, N + 1), axis=1, keepdims=True)
            sel = iota == cand
            selb = sel.astype(jnp.bfloat16)
            gx = (jnp.dot(selb, xh, preferred_element_type=jnp.float32)
                  + jnp.dot(selb, xl, preferred_element_type=jnp.float32)
                  + jnp.dot(selb, xl2,
                            preferred_element_type=jnp.float32))
            feat = jnp.concatenate([gx - xb, xb], axis=1)
            h = jnp.dot(feat, wt, preferred_element_type=jnp.float32)
            return (jnp.where(sel, NEG_INF, d), jnp.maximum(mxb, h),
                    sb + h, qb + h * h)

        init = (db,
                jnp.full((RB, c_out), NEG_INF, jnp.float32),
                jnp.zeros((RB, c_out), jnp.float32),
                jnp.zeros((RB, c_out), jnp.float32))
        _, mxb, sb, qb = lax.fori_loop(0, K, step, init)
        mx_parts.append(mxb)
        p_parts.append(jnp.sum(sb, axis=0))
        q_parts.append(jnp.sum(qb, axis=0))

    hmax_ref[0] = jnp.concatenate(mx_parts, axis=0)
    pso_ref[0] = jnp.sum(jnp.stack(p_parts, 0), axis=0, keepdims=True)
    pqo_ref[0] = jnp.sum(jnp.stack(q_parts, 0), axis=0, keepdims=True)


def _graph_layer(hmax, ps, pq, w, c_in, c_out):
    wt = jnp.transpose(w)
    return pl.pallas_call(
        functools.partial(_graph_body, c_in, c_out),
        grid=(B,),
        in_specs=[
            pl.BlockSpec((1, N, c_in), lambda b: (b, 0, 0)),
            pl.BlockSpec((B, 1, c_in), lambda b: (0, 0, 0)),
            pl.BlockSpec((B, 1, c_in), lambda b: (0, 0, 0)),
            pl.BlockSpec((2 * c_in, c_out), lambda b: (0, 0)),
        ],
        out_specs=[
            pl.BlockSpec((1, N, c_out), lambda b: (b, 0, 0)),
            pl.BlockSpec((1, 1, c_out), lambda b: (b, 0, 0)),
            pl.BlockSpec((1, 1, c_out), lambda b: (b, 0, 0)),
        ],
        out_shape=[
            jax.ShapeDtypeStruct((B, N, c_out), jnp.float32),
            jax.ShapeDtypeStruct((B, 1, c_out), jnp.float32),
            jax.ShapeDtypeStruct((B, 1, c_out), jnp.float32),
        ],
    )(hmax, ps, pq, wt)


def _l5_body(h1_ref, p1_ref, q1_ref, h2_ref, p2_ref, q2_ref,
             h3_ref, p3_ref, q3_ref, h4_ref, p4_ref, q4_ref,
             w_ref, hmax_ref, ps_ref, pq_ref):
    b = pl.program_id(0)
    x1 = _norm_act(h1_ref[0], p1_ref[...], q1_ref[...], CNT_EDGE)
    x2 = _norm_act(h2_ref[0], p2_ref[...], q2_ref[...], CNT_EDGE)
    x3 = _norm_act(h3_ref[0], p3_ref[...], q3_ref[...], CNT_EDGE)
    x4 = _norm_act(h4_ref[0], p4_ref[...], q4_ref[...], CNT_EDGE)
    xc = jnp.concatenate([x1, x2, x3, x4], axis=1)
    h = jnp.dot(xc, w_ref[...], preferred_element_type=jnp.float32)
    hmax_ref[0] = jnp.max(h, axis=0, keepdims=True)
    ps_ref[0] = jnp.sum(h, axis=0, keepdims=True)
    pq_ref[0] = jnp.sum(h * h, axis=0, keepdims=True)


def _layer5(parts, w5):
    w5t = jnp.transpose(w5)
    args = []
    for hm, ps, pq in parts:
        args += [hm, ps, pq]
    args.append(w5t)
    in_specs = []
    for _, ps, _ in parts:
        c = ps.shape[2]
        in_specs += [
            pl.BlockSpec((1, N, c), lambda b: (b, 0, 0)),
            pl.BlockSpec((B, 1, c), lambda b, c=c: (0, 0, 0)),
            pl.BlockSpec((B, 1, c), lambda b, c=c: (0, 0, 0)),
        ]
    in_specs.append(pl.BlockSpec((512, 1024), lambda b: (0, 0)))
    return pl.pallas_call(
        _l5_body,
        grid=(B,),
        in_specs=in_specs,
        out_specs=[
            pl.BlockSpec((1, 1, 1024), lambda b: (b, 0, 0)),
            pl.BlockSpec((1, 1, 1024), lambda b: (b, 0, 0)),
            pl.BlockSpec((1, 1, 1024), lambda b: (b, 0, 0)),
        ],
        out_shape=[
            jax.ShapeDtypeStruct((B, 1, 1024), jnp.float32),
            jax.ShapeDtypeStruct((B, 1, 1024), jnp.float32),
            jax.ShapeDtypeStruct((B, 1, 1024), jnp.float32),
        ],
    )(*args)


def _final_body(hm_ref, ps_ref, pq_ref, out_ref):
    out_ref[...] = _norm_act(hm_ref[:, 0, :], ps_ref[...], pq_ref[...],
                             CNT_PTS)


def _final(hmax5, ps5, pq5):
    return pl.pallas_call(
        _final_body,
        out_shape=jax.ShapeDtypeStruct((B, 1024), jnp.float32),
    )(hmax5, ps5, pq5)


def kernel(x, W1, W2, W3, W4, W5):
    hm1, ps1, pq1 = _layer1(jnp.transpose(x, (0, 3, 2, 1)), jnp.transpose(W1))
    hm2, ps2, pq2 = _graph_layer(hm1, ps1, pq1, W2, 64, 64)
    hm3, ps3, pq3 = _graph_layer(hm2, ps2, pq2, W3, 64, 128)
    hm4, ps4, pq4 = _graph_layer(hm3, ps3, pq3, W4, 128, 256)
    hm5, ps5, pq5 = _layer5(
        [(hm1, ps1, pq1), (hm2, ps2, pq2), (hm3, ps3, pq3), (hm4, ps4, pq4)],
        W5)
    out = _final(hm5, ps5, pq5)
    return out.reshape(B, 1024, 1)


# speedup vs baseline: 5.1292x; 1.6215x over previous
import functools

import jax
import jax.numpy as jnp
from jax import lax
from jax.experimental import pallas as pl

EPS = 1e-5
K = 20
B = 8
N = 1024
CNT_EDGE = float(B * N * K)
CNT_PTS = float(B * N)
NEG_INF = float("-inf")


def _lrelu(x):
    return jnp.where(x >= 0, x, 0.2 * x)


def _norm_act(h, psum, pqsum, cnt):
    m = jnp.sum(psum[:, 0, :], axis=0) / cnt
    v = jnp.sum(pqsum[:, 0, :], axis=0) / cnt - m * m
    sd = jnp.sqrt(v + EPS)
    return _lrelu((h - m) / sd)


def _l1_body(x_ref, w_ref, hmax_ref, ps_ref, pq_ref):
    b = pl.program_id(0)
    k = pl.program_id(1)
    xk = x_ref[0, 0]
    w = w_ref[...]
    h = jnp.dot(xk, w, preferred_element_type=jnp.float32)

    @pl.when(k == 0)
    def _():
        ps_ref[...] = jnp.zeros_like(ps_ref)
        pq_ref[...] = jnp.zeros_like(pq_ref)
        hmax_ref[0] = jnp.full_like(hmax_ref[0], NEG_INF)

    ps_ref[...] += jnp.sum(h, axis=0, keepdims=True)
    pq_ref[...] += jnp.sum(h * h, axis=0, keepdims=True)
    hmax_ref[0] = jnp.maximum(hmax_ref[0], h)


def _layer1(x, w1t):
    return pl.pallas_call(
        _l1_body,
        grid=(B, K),
        in_specs=[
            pl.BlockSpec((1, 1, N, 6), lambda b, k: (b, k, 0, 0)),
            pl.BlockSpec((6, 64), lambda b, k: (0, 0)),
        ],
        out_specs=[
            pl.BlockSpec((1, N, 64), lambda b, k: (b, 0, 0)),
            pl.BlockSpec((1, 1, 64), lambda b, k: (b, 0, 0)),
            pl.BlockSpec((1, 1, 64), lambda b, k: (b, 0, 0)),
        ],
        out_shape=[
            jax.ShapeDtypeStruct((B, N, 64), jnp.float32),
            jax.ShapeDtypeStruct((B, 1, 64), jnp.float32),
            jax.ShapeDtypeStruct((B, 1, 64), jnp.float32),
        ],
    )(x, w1t)


RB = 128
NRB = N // RB


def _graph_body(c_in, c_out, hp_ref, ps_ref, pq_ref, wt_ref,
                hmax_ref, pso_ref, pqo_ref):
    b = pl.program_id(0)
    xin = _norm_act(hp_ref[0], ps_ref[...], pq_ref[...], CNT_EDGE)
    wt = wt_ref[...]
    xh = xin.astype(jnp.bfloat16)
    r1 = xin - xh.astype(jnp.float32)
    xl = r1.astype(jnp.bfloat16)
    xl2 = (r1 - xl.astype(jnp.float32)).astype(jnp.bfloat16)
    xx = jnp.sum(xin * xin, axis=1)
    xrow = xx[None, :]
    iota = lax.broadcasted_iota(jnp.int32, (RB, N), 1)

    mx_parts = []
    p_parts = []
    q_parts = []
    for rb in range(NRB):
        sl = slice(rb * RB, (rb + 1) * RB)
        xb = xin[sl]
        gb = lax.dot_general(xb, xin, (((1,), (1,)), ((), ())),
                             preferred_element_type=jnp.float32)
        inner = -2.0 * gb
        db = (-xx[sl][:, None] - inner) - xrow

        def step(_, carry):
            d, mxb, sb, qb = carry
            v = jnp.max(d, axis=1, keepdims=True)
            eq = d == v
            cand = jnp.min(jnp.where(eq, iota, N + 1), axis=1, keepdims=True)
            sel = iota == cand
            selb = sel.astype(jnp.bfloat16)
            gx = (jnp.dot(selb, xh, preferred_element_type=jnp.float32)
                  + jnp.dot(selb, xl, preferred_element_type=jnp.float32)
                  + jnp.dot(selb, xl2,
                            preferred_element_type=jnp.float32))
            feat = jnp.concatenate([gx - xb, xb], axis=1)
            h = jnp.dot(feat, wt, preferred_element_type=jnp.float32)
            return (jnp.where(sel, NEG_INF, d), jnp.maximum(mxb, h),
                    sb + h, qb + h * h)

        init = (db,
                jnp.full((RB, c_out), NEG_INF, jnp.float32),
                jnp.zeros((RB, c_out), jnp.float32),
                jnp.zeros((RB, c_out), jnp.float32))
        _, mxb, sb, qb = lax.fori_loop(0, K, step, init, unroll=4)
        mx_parts.append(mxb)
        p_parts.append(jnp.sum(sb, axis=0))
        q_parts.append(jnp.sum(qb, axis=0))

    hmax_ref[0] = jnp.concatenate(mx_parts, axis=0)
    pso_ref[0] = jnp.sum(jnp.stack(p_parts, 0), axis=0, keepdims=True)
    pqo_ref[0] = jnp.sum(jnp.stack(q_parts, 0), axis=0, keepdims=True)


def _graph_layer(hmax, ps, pq, w, c_in, c_out):
    wt = jnp.transpose(w)
    return pl.pallas_call(
        functools.partial(_graph_body, c_in, c_out),
        grid=(B,),
        in_specs=[
            pl.BlockSpec((1, N, c_in), lambda b: (b, 0, 0)),
            pl.BlockSpec((B, 1, c_in), lambda b: (0, 0, 0)),
            pl.BlockSpec((B, 1, c_in), lambda b: (0, 0, 0)),
            pl.BlockSpec((2 * c_in, c_out), lambda b: (0, 0)),
        ],
        out_specs=[
            pl.BlockSpec((1, N, c_out), lambda b: (b, 0, 0)),
            pl.BlockSpec((1, 1, c_out), lambda b: (b, 0, 0)),
            pl.BlockSpec((1, 1, c_out), lambda b: (b, 0, 0)),
        ],
        out_shape=[
            jax.ShapeDtypeStruct((B, N, c_out), jnp.float32),
            jax.ShapeDtypeStruct((B, 1, c_out), jnp.float32),
            jax.ShapeDtypeStruct((B, 1, c_out), jnp.float32),
        ],
    )(hmax, ps, pq, wt)


def _l5_body(h1_ref, p1_ref, q1_ref, h2_ref, p2_ref, q2_ref,
             h3_ref, p3_ref, q3_ref, h4_ref, p4_ref, q4_ref,
             w_ref, hmax_ref, ps_ref, pq_ref):
    b = pl.program_id(0)
    x1 = _norm_act(h1_ref[0], p1_ref[...], q1_ref[...], CNT_EDGE)
    x2 = _norm_act(h2_ref[0], p2_ref[...], q2_ref[...], CNT_EDGE)
    x3 = _norm_act(h3_ref[0], p3_ref[...], q3_ref[...], CNT_EDGE)
    x4 = _norm_act(h4_ref[0], p4_ref[...], q4_ref[...], CNT_EDGE)
    xc = jnp.concatenate([x1, x2, x3, x4], axis=1)
    h = jnp.dot(xc, w_ref[...], preferred_element_type=jnp.float32)
    hmax_ref[0] = jnp.max(h, axis=0, keepdims=True)
    ps_ref[0] = jnp.sum(h, axis=0, keepdims=True)
    pq_ref[0] = jnp.sum(h * h, axis=0, keepdims=True)


def _layer5(parts, w5):
    w5t = jnp.transpose(w5)
    args = []
    for hm, ps, pq in parts:
        args += [hm, ps, pq]
    args.append(w5t)
    in_specs = []
    for _, ps, _ in parts:
        c = ps.shape[2]
        in_specs += [
            pl.BlockSpec((1, N, c), lambda b: (b, 0, 0)),
            pl.BlockSpec((B, 1, c), lambda b, c=c: (0, 0, 0)),
            pl.BlockSpec((B, 1, c), lambda b, c=c: (0, 0, 0)),
        ]
    in_specs.append(pl.BlockSpec((512, 1024), lambda b: (0, 0)))
    return pl.pallas_call(
        _l5_body,
        grid=(B,),
        in_specs=in_specs,
        out_specs=[
            pl.BlockSpec((1, 1, 1024), lambda b: (b, 0, 0)),
            pl.BlockSpec((1, 1, 1024), lambda b: (b, 0, 0)),
            pl.BlockSpec((1, 1, 1024), lambda b: (b, 0, 0)),
        ],
        out_shape=[
            jax.ShapeDtypeStruct((B, 1, 1024), jnp.float32),
            jax.ShapeDtypeStruct((B, 1, 1024), jnp.float32),
            jax.ShapeDtypeStruct((B, 1, 1024), jnp.float32),
        ],
    )(*args)


def _final_body(hm_ref, ps_ref, pq_ref, out_ref):
    out_ref[...] = _norm_act(hm_ref[:, 0, :], ps_ref[...], pq_ref[...],
                             CNT_PTS)


def _final(hmax5, ps5, pq5):
    return pl.pallas_call(
        _final_body,
        out_shape=jax.ShapeDtypeStruct((B, 1024), jnp.float32),
    )(hmax5, ps5, pq5)


def kernel(x, W1, W2, W3, W4, W5):
    hm1, ps1, pq1 = _layer1(jnp.transpose(x, (0, 3, 2, 1)), jnp.transpose(W1))
    hm2, ps2, pq2 = _graph_layer(hm1, ps1, pq1, W2, 64, 64)
    hm3, ps3, pq3 = _graph_layer(hm2, ps2, pq2, W3, 64, 128)
    hm4, ps4, pq4 = _graph_layer(hm3, ps3, pq3, W4, 128, 256)
    hm5, ps5, pq5 = _layer5(
        [(hm1, ps1, pq1), (hm2, ps2, pq2), (hm3, ps3, pq3), (hm4, ps4, pq4)],
        W5)
    out = _final(hm5, ps5, pq5)
    return out.reshape(B, 1024, 1)


# speedup vs baseline: 5.6721x; 1.1059x over previous
import functools

import jax
import jax.numpy as jnp
from jax import lax
from jax.experimental import pallas as pl

EPS = 1e-5
K = 20
B = 8
N = 1024
CNT_EDGE = float(B * N * K)
CNT_PTS = float(B * N)
NEG_INF = float("-inf")


def _lrelu(x):
    return jnp.where(x >= 0, x, 0.2 * x)


def _norm_act(h, psum, pqsum, cnt):
    m = jnp.sum(psum[:, 0, :], axis=0) / cnt
    v = jnp.sum(pqsum[:, 0, :], axis=0) / cnt - m * m
    sd = jnp.sqrt(v + EPS)
    return _lrelu((h - m) / sd)


def _l1_body(x_ref, w_ref, hmax_ref, ps_ref, pq_ref):
    b = pl.program_id(0)
    k = pl.program_id(1)
    xk = x_ref[0, 0]
    w = w_ref[...]
    h = jnp.dot(xk, w, preferred_element_type=jnp.float32)

    @pl.when(k == 0)
    def _():
        ps_ref[...] = jnp.zeros_like(ps_ref)
        pq_ref[...] = jnp.zeros_like(pq_ref)
        hmax_ref[0] = jnp.full_like(hmax_ref[0], NEG_INF)

    ps_ref[...] += jnp.sum(h, axis=0, keepdims=True)
    pq_ref[...] += jnp.sum(h * h, axis=0, keepdims=True)
    hmax_ref[0] = jnp.maximum(hmax_ref[0], h)


def _layer1(x, w1t):
    return pl.pallas_call(
        _l1_body,
        grid=(B, K),
        in_specs=[
            pl.BlockSpec((1, 1, N, 6), lambda b, k: (b, k, 0, 0)),
            pl.BlockSpec((6, 64), lambda b, k: (0, 0)),
        ],
        out_specs=[
            pl.BlockSpec((1, N, 64), lambda b, k: (b, 0, 0)),
            pl.BlockSpec((1, 1, 64), lambda b, k: (b, 0, 0)),
            pl.BlockSpec((1, 1, 64), lambda b, k: (b, 0, 0)),
        ],
        out_shape=[
            jax.ShapeDtypeStruct((B, N, 64), jnp.float32),
            jax.ShapeDtypeStruct((B, 1, 64), jnp.float32),
            jax.ShapeDtypeStruct((B, 1, 64), jnp.float32),
        ],
    )(x, w1t)


RB = 128
NRB = N // RB


def _graph_body(c_in, c_out, hp_ref, ps_ref, pq_ref, wt_ref,
                hmax_ref, pso_ref, pqo_ref):
    b = pl.program_id(0)
    xin = _norm_act(hp_ref[0], ps_ref[...], pq_ref[...], CNT_EDGE)
    wt = wt_ref[...]
    xh = xin.astype(jnp.bfloat16)
    r1 = xin - xh.astype(jnp.float32)
    xl = r1.astype(jnp.bfloat16)
    xl2 = (r1 - xl.astype(jnp.float32)).astype(jnp.bfloat16)
    xx = jnp.sum(xin * xin, axis=1)
    xrow = xx[None, :]
    iota = lax.broadcasted_iota(jnp.int32, (RB, N), 1)

    mx_parts = []
    p_parts = []
    q_parts = []
    for rb in range(NRB):
        sl = slice(rb * RB, (rb + 1) * RB)
        xb = xin[sl]
        gb = lax.dot_general(xb, xin, (((1,), (1,)), ((), ())),
                             preferred_element_type=jnp.float32)
        inner = -2.0 * gb
        db = (-xx[sl][:, None] - inner) - xrow

        def step(_, carry):
            d, mxb, sb, qb = carry
            v = jnp.max(d, axis=1, keepdims=True)
            eq = d == v
            cand = jnp.min(jnp.where(eq, iota, N + 1), axis=1, keepdims=True)
            sel = iota == cand
            selb = sel.astype(jnp.bfloat16)
            gx = (jnp.dot(selb, xh, preferred_element_type=jnp.float32)
                  + jnp.dot(selb, xl, preferred_element_type=jnp.float32)
                  + jnp.dot(selb, xl2,
                            preferred_element_type=jnp.float32))
            feat = jnp.concatenate([gx - xb, xb], axis=1)
            h = jnp.dot(feat, wt, preferred_element_type=jnp.float32)
            return (jnp.where(sel, NEG_INF, d), jnp.maximum(mxb, h),
                    sb + h, qb + h * h)

        init = (db,
                jnp.full((RB, c_out), NEG_INF, jnp.float32),
                jnp.zeros((RB, c_out), jnp.float32),
                jnp.zeros((RB, c_out), jnp.float32))
        _, mxb, sb, qb = lax.fori_loop(0, K, step, init, unroll=10)
        mx_parts.append(mxb)
        p_parts.append(jnp.sum(sb, axis=0))
        q_parts.append(jnp.sum(qb, axis=0))

    hmax_ref[0] = jnp.concatenate(mx_parts, axis=0)
    pso_ref[0] = jnp.sum(jnp.stack(p_parts, 0), axis=0, keepdims=True)
    pqo_ref[0] = jnp.sum(jnp.stack(q_parts, 0), axis=0, keepdims=True)


def _graph_layer(hmax, ps, pq, w, c_in, c_out):
    wt = jnp.transpose(w)
    return pl.pallas_call(
        functools.partial(_graph_body, c_in, c_out),
        grid=(B,),
        in_specs=[
            pl.BlockSpec((1, N, c_in), lambda b: (b, 0, 0)),
            pl.BlockSpec((B, 1, c_in), lambda b: (0, 0, 0)),
            pl.BlockSpec((B, 1, c_in), lambda b: (0, 0, 0)),
            pl.BlockSpec((2 * c_in, c_out), lambda b: (0, 0)),
        ],
        out_specs=[
            pl.BlockSpec((1, N, c_out), lambda b: (b, 0, 0)),
            pl.BlockSpec((1, 1, c_out), lambda b: (b, 0, 0)),
            pl.BlockSpec((1, 1, c_out), lambda b: (b, 0, 0)),
        ],
        out_shape=[
            jax.ShapeDtypeStruct((B, N, c_out), jnp.float32),
            jax.ShapeDtypeStruct((B, 1, c_out), jnp.float32),
            jax.ShapeDtypeStruct((B, 1, c_out), jnp.float32),
        ],
    )(hmax, ps, pq, wt)


def _l5_body(h1_ref, p1_ref, q1_ref, h2_ref, p2_ref, q2_ref,
             h3_ref, p3_ref, q3_ref, h4_ref, p4_ref, q4_ref,
             w_ref, hmax_ref, ps_ref, pq_ref):
    b = pl.program_id(0)
    x1 = _norm_act(h1_ref[0], p1_ref[...], q1_ref[...], CNT_EDGE)
    x2 = _norm_act(h2_ref[0], p2_ref[...], q2_ref[...], CNT_EDGE)
    x3 = _norm_act(h3_ref[0], p3_ref[...], q3_ref[...], CNT_EDGE)
    x4 = _norm_act(h4_ref[0], p4_ref[...], q4_ref[...], CNT_EDGE)
    xc = jnp.concatenate([x1, x2, x3, x4], axis=1)
    h = jnp.dot(xc, w_ref[...], preferred_element_type=jnp.float32)
    hmax_ref[0] = jnp.max(h, axis=0, keepdims=True)
    ps_ref[0] = jnp.sum(h, axis=0, keepdims=True)
    pq_ref[0] = jnp.sum(h * h, axis=0, keepdims=True)


def _layer5(parts, w5):
    w5t = jnp.transpose(w5)
    args = []
    for hm, ps, pq in parts:
        args += [hm, ps, pq]
    args.append(w5t)
    in_specs = []
    for _, ps, _ in parts:
        c = ps.shape[2]
        in_specs += [
            pl.BlockSpec((1, N, c), lambda b: (b, 0, 0)),
            pl.BlockSpec((B, 1, c), lambda b, c=c: (0, 0, 0)),
            pl.BlockSpec((B, 1, c), lambda b, c=c: (0, 0, 0)),
        ]
    in_specs.append(pl.BlockSpec((512, 1024), lambda b: (0, 0)))
    return pl.pallas_call(
        _l5_body,
        grid=(B,),
        in_specs=in_specs,
        out_specs=[
            pl.BlockSpec((1, 1, 1024), lambda b: (b, 0, 0)),
            pl.BlockSpec((1, 1, 1024), lambda b: (b, 0, 0)),
            pl.BlockSpec((1, 1, 1024), lambda b: (b, 0, 0)),
        ],
        out_shape=[
            jax.ShapeDtypeStruct((B, 1, 1024), jnp.float32),
            jax.ShapeDtypeStruct((B, 1, 1024), jnp.float32),
            jax.ShapeDtypeStruct((B, 1, 1024), jnp.float32),
        ],
    )(*args)


def _final_body(hm_ref, ps_ref, pq_ref, out_ref):
    out_ref[...] = _norm_act(hm_ref[:, 0, :], ps_ref[...], pq_ref[...],
                             CNT_PTS)


def _final(hmax5, ps5, pq5):
    return pl.pallas_call(
        _final_body,
        out_shape=jax.ShapeDtypeStruct((B, 1024), jnp.float32),
    )(hmax5, ps5, pq5)


def kernel(x, W1, W2, W3, W4, W5):
    hm1, ps1, pq1 = _layer1(jnp.transpose(x, (0, 3, 2, 1)), jnp.transpose(W1))
    hm2, ps2, pq2 = _graph_layer(hm1, ps1, pq1, W2, 64, 64)
    hm3, ps3, pq3 = _graph_layer(hm2, ps2, pq2, W3, 64, 128)
    hm4, ps4, pq4 = _graph_layer(hm3, ps3, pq3, W4, 128, 256)
    hm5, ps5, pq5 = _layer5(
        [(hm1, ps1, pq1), (hm2, ps2, pq2), (hm3, ps3, pq3), (hm4, ps4, pq4)],
        W5)
    out = _final(hm5, ps5, pq5)
    return out.reshape(B, 1024, 1)


# speedup vs baseline: 9.5790x; 1.6888x over previous
import functools

import jax
import jax.numpy as jnp
from jax import lax
from jax.experimental import pallas as pl
from jax.experimental.pallas import tpu as pltpu
from jax.experimental.pallas import tpu_sc as plsc

EPS = 1e-5
K = 20
B = 8
N = 1024
CNT_EDGE = float(B * N * K)
CNT_PTS = float(B * N)
NEG_INF = float("-inf")


def _lrelu(x):
    return jnp.where(x >= 0, x, 0.2 * x)


def _norm_act(h, psum, pqsum, cnt):
    m = jnp.sum(psum[:, 0, :], axis=0) / cnt
    v = jnp.sum(pqsum[:, 0, :], axis=0) / cnt - m * m
    sd = jnp.sqrt(v + EPS)
    return _lrelu((h - m) / sd)


def _l1_body(x_ref, w_ref, hmax_ref, ps_ref, pq_ref):
    b = pl.program_id(0)
    k = pl.program_id(1)
    xk = x_ref[0, 0]
    w = w_ref[...]
    h = jnp.dot(xk, w, preferred_element_type=jnp.float32)

    @pl.when(k == 0)
    def _():
        ps_ref[...] = jnp.zeros_like(ps_ref)
        pq_ref[...] = jnp.zeros_like(pq_ref)
        hmax_ref[0] = jnp.full_like(hmax_ref[0], NEG_INF)

    ps_ref[...] += jnp.sum(h, axis=0, keepdims=True)
    pq_ref[...] += jnp.sum(h * h, axis=0, keepdims=True)
    hmax_ref[0] = jnp.maximum(hmax_ref[0], h)


def _layer1(x, w1t):
    return pl.pallas_call(
        _l1_body,
        grid=(B, K),
        in_specs=[
            pl.BlockSpec((1, 1, N, 6), lambda b, k: (b, k, 0, 0)),
            pl.BlockSpec((6, 64), lambda b, k: (0, 0)),
        ],
        out_specs=[
            pl.BlockSpec((1, N, 64), lambda b, k: (b, 0, 0)),
            pl.BlockSpec((1, 1, 64), lambda b, k: (b, 0, 0)),
            pl.BlockSpec((1, 1, 64), lambda b, k: (b, 0, 0)),
        ],
        out_shape=[
            jax.ShapeDtypeStruct((B, N, 64), jnp.float32),
            jax.ShapeDtypeStruct((B, 1, 64), jnp.float32),
            jax.ShapeDtypeStruct((B, 1, 64), jnp.float32),
        ],
    )(x, w1t)


RB = 128
NRB = N // RB


def _graph_body(c_in, c_out, hp_ref, ps_ref, pq_ref, wt_ref,
                hmax_ref, pso_ref, pqo_ref):
    b = pl.program_id(0)
    xin = _norm_act(hp_ref[0], ps_ref[...], pq_ref[...], CNT_EDGE)
    wt = wt_ref[...]
    xh = xin.astype(jnp.bfloat16)
    r1 = xin - xh.astype(jnp.float32)
    xl = r1.astype(jnp.bfloat16)
    xl2 = (r1 - xl.astype(jnp.float32)).astype(jnp.bfloat16)
    xx = jnp.sum(xin * xin, axis=1)
    xrow = xx[None, :]
    iota = lax.broadcasted_iota(jnp.int32, (RB, N), 1)

    mx_parts = []
    p_parts = []
    q_parts = []
    for rb in range(NRB):
        sl = slice(rb * RB, (rb + 1) * RB)
        xb = xin[sl]
        gb = lax.dot_general(xb, xin, (((1,), (1,)), ((), ())),
                             preferred_element_type=jnp.float32)
        inner = -2.0 * gb
        db = (-xx[sl][:, None] - inner) - xrow

        def step(_, carry):
            d, mxb, sb, qb = carry
            v = jnp.max(d, axis=1, keepdims=True)
            eq = d == v
            cand = jnp.min(jnp.where(eq, iota, N + 1), axis=1, keepdims=True)
            sel = iota == cand
            selb = sel.astype(jnp.bfloat16)
            gx = (jnp.dot(selb, xh, preferred_element_type=jnp.float32)
                  + jnp.dot(selb, xl, preferred_element_type=jnp.float32)
                  + jnp.dot(selb, xl2,
                            preferred_element_type=jnp.float32))
            feat = jnp.concatenate([gx - xb, xb], axis=1)
            h = jnp.dot(feat, wt, preferred_element_type=jnp.float32)
            return (jnp.where(sel, NEG_INF, d), jnp.maximum(mxb, h),
                    sb + h, qb + h * h)

        init = (db,
                jnp.full((RB, c_out), NEG_INF, jnp.float32),
                jnp.zeros((RB, c_out), jnp.float32),
                jnp.zeros((RB, c_out), jnp.float32))
        _, mxb, sb, qb = lax.fori_loop(0, K, step, init, unroll=10)
        mx_parts.append(mxb)
        p_parts.append(jnp.sum(sb, axis=0))
        q_parts.append(jnp.sum(qb, axis=0))

    hmax_ref[0] = jnp.concatenate(mx_parts, axis=0)
    pso_ref[0] = jnp.sum(jnp.stack(p_parts, 0), axis=0, keepdims=True)
    pqo_ref[0] = jnp.sum(jnp.stack(q_parts, 0), axis=0, keepdims=True)


def _sel_body(c_in, hp_ref, ps_ref, pq_ref, xn_ref, idx_ref):
    b = pl.program_id(0)
    xin = _norm_act(hp_ref[0], ps_ref[...], pq_ref[...], CNT_EDGE)
    xn_ref[0] = xin
    xx = jnp.sum(xin * xin, axis=1)
    xrow = xx[None, :]
    iota = lax.broadcasted_iota(jnp.int32, (RB, N), 1)
    for rb in range(NRB):
        sl = slice(rb * RB, (rb + 1) * RB)
        xb = xin[sl]
        gb = lax.dot_general(xb, xin, (((1,), (1,)), ((), ())),
                             preferred_element_type=jnp.float32)
        inner = -2.0 * gb
        d = (-xx[sl][:, None] - inner) - xrow
        for s in range(K):
            v = jnp.max(d, axis=1, keepdims=True)
            eq = d == v
            cand = jnp.min(jnp.where(eq, iota, N + 1), axis=1, keepdims=True)
            sel = iota == cand
            idx_ref[0, sl, s] = cand[:, 0] + b * N
            d = jnp.where(sel, NEG_INF, d)


def _gather_body(x_hbm, idx_hbm, out_hbm, idx_v, rows_v, sem):
    wid = lax.axis_index("s") * 2 + lax.axis_index("c")
    nrow = (B * N * K) // 32
    chunk = 640
    for ch in range(nrow // chunk):
        off = wid * nrow + ch * chunk
        pltpu.sync_copy(idx_hbm.at[pl.ds(off, chunk)], idx_v)
        pltpu.async_copy(x_hbm.at[idx_v], rows_v, sem).wait()
        pltpu.sync_copy(rows_v, out_hbm.at[pl.ds(off, chunk)])


def _conv_body(c_in, c_out, xn_ref, gx_ref, wt_ref, hmax_ref, pso_ref,
               pqo_ref):
    xb = xn_ref[0]
    wt = wt_ref[...]
    mx = jnp.full((N, c_out), NEG_INF, jnp.float32)
    sb = jnp.zeros((N, c_out), jnp.float32)
    qb = jnp.zeros((N, c_out), jnp.float32)
    for k in range(K):
        gk = gx_ref[0, k][:, :c_in]
        feat = jnp.concatenate([gk - xb, xb], axis=1)
        h = jnp.dot(feat, wt, preferred_element_type=jnp.float32)
        mx = jnp.maximum(mx, h)
        sb = sb + h
        qb = qb + h * h
    hmax_ref[0] = mx
    pso_ref[0] = jnp.sum(sb, axis=0, keepdims=True)
    pqo_ref[0] = jnp.sum(qb, axis=0, keepdims=True)


def _graph_layer_sc(hmax, ps, pq, w, c_in, c_out):
    wt = jnp.transpose(w)
    xn, idxp = pl.pallas_call(
        functools.partial(_sel_body, c_in),
        grid=(B,),
        in_specs=[
            pl.BlockSpec((1, N, c_in), lambda b: (b, 0, 0)),
            pl.BlockSpec((B, 1, c_in), lambda b: (0, 0, 0)),
            pl.BlockSpec((B, 1, c_in), lambda b: (0, 0, 0)),
        ],
        out_specs=[
            pl.BlockSpec((1, N, c_in), lambda b: (b, 0, 0)),
            pl.BlockSpec((1, N, 128), lambda b: (b, 0, 0)),
        ],
        out_shape=[
            jax.ShapeDtypeStruct((B, N, c_in), jnp.float32),
            jax.ShapeDtypeStruct((B, N, 128), jnp.int32),
        ],
    )(hmax, ps, pq)
    idx_flat = jnp.transpose(idxp[:, :, :K], (0, 2, 1)).reshape(-1)
    cd = max(c_in, 128)
    tbl = xn.reshape(B * N, c_in)
    if cd != c_in:
        tbl = jnp.pad(tbl, ((0, 0), (0, cd - c_in)))
    gather = functools.partial(
        pl.kernel,
        out_type=jax.ShapeDtypeStruct((B * N * K, cd), jnp.float32),
        mesh=plsc.VectorSubcoreMesh(core_axis_name="c", subcore_axis_name="s"),
        scratch_types=[
            pltpu.VMEM((640,), jnp.int32),
            pltpu.VMEM((640, cd), jnp.float32),
            pltpu.SemaphoreType.DMA,
        ],
    )(_gather_body)
    gx = gather(tbl, idx_flat)
    gx = gx.reshape(B, K, N, cd)
    return pl.pallas_call(
        functools.partial(_conv_body, c_in, c_out),
        grid=(B,),
        in_specs=[
            pl.BlockSpec((1, N, c_in), lambda b: (b, 0, 0)),
            pl.BlockSpec((1, K, N, cd), lambda b: (b, 0, 0, 0)),
            pl.BlockSpec((2 * c_in, c_out), lambda b: (0, 0)),
        ],
        out_specs=[
            pl.BlockSpec((1, N, c_out), lambda b: (b, 0, 0)),
            pl.BlockSpec((1, 1, c_out), lambda b: (b, 0, 0)),
            pl.BlockSpec((1, 1, c_out), lambda b: (b, 0, 0)),
        ],
        out_shape=[
            jax.ShapeDtypeStruct((B, N, c_out), jnp.float32),
            jax.ShapeDtypeStruct((B, 1, c_out), jnp.float32),
            jax.ShapeDtypeStruct((B, 1, c_out), jnp.float32),
        ],
    )(xn, gx, wt)


def _graph_layer(hmax, ps, pq, w, c_in, c_out):
    wt = jnp.transpose(w)
    return pl.pallas_call(
        functools.partial(_graph_body, c_in, c_out),
        grid=(B,),
        in_specs=[
            pl.BlockSpec((1, N, c_in), lambda b: (b, 0, 0)),
            pl.BlockSpec((B, 1, c_in), lambda b: (0, 0, 0)),
            pl.BlockSpec((B, 1, c_in), lambda b: (0, 0, 0)),
            pl.BlockSpec((2 * c_in, c_out), lambda b: (0, 0)),
        ],
        out_specs=[
            pl.BlockSpec((1, N, c_out), lambda b: (b, 0, 0)),
            pl.BlockSpec((1, 1, c_out), lambda b: (b, 0, 0)),
            pl.BlockSpec((1, 1, c_out), lambda b: (b, 0, 0)),
        ],
        out_shape=[
            jax.ShapeDtypeStruct((B, N, c_out), jnp.float32),
            jax.ShapeDtypeStruct((B, 1, c_out), jnp.float32),
            jax.ShapeDtypeStruct((B, 1, c_out), jnp.float32),
        ],
    )(hmax, ps, pq, wt)


def _l5_body(h1_ref, p1_ref, q1_ref, h2_ref, p2_ref, q2_ref,
             h3_ref, p3_ref, q3_ref, h4_ref, p4_ref, q4_ref,
             w_ref, hmax_ref, ps_ref, pq_ref):
    b = pl.program_id(0)
    x1 = _norm_act(h1_ref[0], p1_ref[...], q1_ref[...], CNT_EDGE)
    x2 = _norm_act(h2_ref[0], p2_ref[...], q2_ref[...], CNT_EDGE)
    x3 = _norm_act(h3_ref[0], p3_ref[...], q3_ref[...], CNT_EDGE)
    x4 = _norm_act(h4_ref[0], p4_ref[...], q4_ref[...], CNT_EDGE)
    xc = jnp.concatenate([x1, x2, x3, x4], axis=1)
    h = jnp.dot(xc, w_ref[...], preferred_element_type=jnp.float32)
    hmax_ref[0] = jnp.max(h, axis=0, keepdims=True)
    ps_ref[0] = jnp.sum(h, axis=0, keepdims=True)
    pq_ref[0] = jnp.sum(h * h, axis=0, keepdims=True)


def _layer5(parts, w5):
    w5t = jnp.transpose(w5)
    args = []
    for hm, ps, pq in parts:
        args += [hm, ps, pq]
    args.append(w5t)
    in_specs = []
    for _, ps, _ in parts:
        c = ps.shape[2]
        in_specs += [
            pl.BlockSpec((1, N, c), lambda b: (b, 0, 0)),
            pl.BlockSpec((B, 1, c), lambda b, c=c: (0, 0, 0)),
            pl.BlockSpec((B, 1, c), lambda b, c=c: (0, 0, 0)),
        ]
    in_specs.append(pl.BlockSpec((512, 1024), lambda b: (0, 0)))
    return pl.pallas_call(
        _l5_body,
        grid=(B,),
        in_specs=in_specs,
        out_specs=[
            pl.BlockSpec((1, 1, 1024), lambda b: (b, 0, 0)),
            pl.BlockSpec((1, 1, 1024), lambda b: (b, 0, 0)),
            pl.BlockSpec((1, 1, 1024), lambda b: (b, 0, 0)),
        ],
        out_shape=[
            jax.ShapeDtypeStruct((B, 1, 1024), jnp.float32),
            jax.ShapeDtypeStruct((B, 1, 1024), jnp.float32),
            jax.ShapeDtypeStruct((B, 1, 1024), jnp.float32),
        ],
    )(*args)


def _final_body(hm_ref, ps_ref, pq_ref, out_ref):
    out_ref[...] = _norm_act(hm_ref[:, 0, :], ps_ref[...], pq_ref[...],
                             CNT_PTS)


def _final(hmax5, ps5, pq5):
    return pl.pallas_call(
        _final_body,
        out_shape=jax.ShapeDtypeStruct((B, 1024), jnp.float32),
    )(hmax5, ps5, pq5)


def kernel(x, W1, W2, W3, W4, W5):
    hm1, ps1, pq1 = _layer1(jnp.transpose(x, (0, 3, 2, 1)), jnp.transpose(W1))
    hm2, ps2, pq2 = _graph_layer_sc(hm1, ps1, pq1, W2, 64, 64)
    hm3, ps3, pq3 = _graph_layer_sc(hm2, ps2, pq2, W3, 64, 128)
    hm4, ps4, pq4 = _graph_layer_sc(hm3, ps3, pq3, W4, 128, 256)
    hm5, ps5, pq5 = _layer5(
        [(hm1, ps1, pq1), (hm2, ps2, pq2), (hm3, ps3, pq3), (hm4, ps4, pq4)],
        W5)
    out = _final(hm5, ps5, pq5)
    return out.reshape(B, 1024, 1)
